# Initial kernel scaffold; baseline (speedup 1.0000x reference)
import jax
import jax.numpy as jnp
from jax.experimental import pallas as pl


def kernel(f_distribution, collision_term, source_term, edge_weight, edge_index):
    raise NotImplementedError("write your pallas kernel here")


# speedup vs baseline: 3.1634x; 3.1634x over previous
import functools

import jax
import jax.numpy as jnp
from jax import lax
from jax.experimental import pallas as pl
from jax.experimental.pallas import tpu as pltpu
from jax.experimental.pallas import tpu_sc as plsc

N = 10000
E = 320000
Q = 128
DT = 0.1
MAX_XI = 75.0

NC = 2
NS = 16
NW = NC * NS
QPW = Q // NW
NPW = QPW * N
K = 2000
NCH = E // K
IPV = K // 16


@functools.partial(
    pl.kernel,
    mesh=plsc.VectorSubcoreMesh(core_axis_name="c", subcore_axis_name="s"),
    out_type=jax.ShapeDtypeStruct((Q * N,), jnp.float32),
    compiler_params=pltpu.CompilerParams(needs_layout_passes=False),
    scratch_types=[
        pltpu.VMEM((NPW,), jnp.float32),
        pltpu.VMEM((NPW,), jnp.float32),
        pltpu.VMEM((N,), jnp.float32),
        pltpu.VMEM((N,), jnp.float32),
        pltpu.VMEM((2 * K,), jnp.int32),
        pltpu.VMEM((2 * K,), jnp.int32),
        pltpu.VMEM((2 * K,), jnp.float32),
        pltpu.VMEM((K,), jnp.float32),
        pltpu.VMEM((K,), jnp.float32),
        pltpu.VMEM((K,), jnp.float32),
        pltpu.SemaphoreType.DMA,
        pltpu.SemaphoreType.DMA,
        pltpu.SemaphoreType.DMA,
        pltpu.SemaphoreType.DMA,
        pltpu.SemaphoreType.DMA,
        pltpu.SemaphoreType.DMA,
    ],
)
def _sc_step(fT, collT, srcT, w_hbm, src_hbm, dst_hbm, out,
             f_v, acc_v, deg_v, s_v, esrc, edst, ew, cb, sb, ob,
             sem_s0, sem_s1, sem_d0, sem_d1, sem_w0, sem_w1):
    wid = lax.axis_index("s") * NC + lax.axis_index("c")
    base = wid * NPW
    sem_s = (sem_s0, sem_s1)
    sem_d = (sem_d0, sem_d1)
    sem_w = (sem_w0, sem_w1)

    zeros = jnp.zeros((16,), jnp.float32)
    ones = jnp.full((16,), 1.0, jnp.float32)

    pltpu.sync_copy(fT.at[pl.ds(base, NPW)], f_v)

    def _init_f(i, c):
        sl = pl.ds(i * 16, 16)
        f_v[sl] = jnp.maximum(f_v[sl], 0.0)
        acc_v[sl] = zeros
        return c

    lax.fori_loop(0, NPW // 16, _init_f, 0)

    def _init_n(i, c):
        sl = pl.ds(i * 16, 16)
        deg_v[sl] = zeros
        s_v[sl] = zeros
        return c

    lax.fori_loop(0, N // 16, _init_n, 0)

    def _pipelined(start, process):
        start(0, 0)

        def body(g, carry):
            j0 = g * 2
            start(j0 + 1, 1)
            _wait(j0, 0)
            process(0)

            @pl.when(g + 1 < NCH // 2)
            def _():
                start(j0 + 2, 0)

            _wait(j0 + 1, 1)
            process(1)
            return carry

        lax.fori_loop(0, NCH // 2, body, 0)

    def _copies(j, slot):
        sl_h = pl.ds(j * K, K)
        sl_v = pl.ds(slot * K, K)
        return (
            pltpu.make_async_copy(src_hbm.at[sl_h], esrc.at[sl_v], sem_s[slot]),
            pltpu.make_async_copy(dst_hbm.at[sl_h], edst.at[sl_v], sem_d[slot]),
            pltpu.make_async_copy(w_hbm.at[sl_h], ew.at[sl_v], sem_w[slot]),
        )

    def _start(j, slot):
        for c in _copies(j, slot):
            c.start()

    def _wait(j, slot):
        for c in _copies(j, slot):
            c.wait()

    def _p1_start(j, slot):
        pltpu.make_async_copy(src_hbm.at[pl.ds(j * K, K)],
                              esrc.at[pl.ds(slot * K, K)], sem_s[slot]).start()

    def _p1_proc(slot):
        def ib(i, c):
            idx = esrc[pl.ds(slot * K + i * 16, 16)]
            plsc.addupdate_scatter(deg_v, [idx], ones)
            return c

        lax.fori_loop(0, IPV, ib, 0)

    def _p1_pipelined():
        _p1_start(0, 0)

        def body(g, carry):
            j0 = g * 2
            _p1_start(j0 + 1, 1)
            pltpu.make_async_copy(src_hbm.at[pl.ds(j0 * K, K)],
                                  esrc.at[pl.ds(0, K)], sem_s[0]).wait()
            _p1_proc(0)

            @pl.when(g + 1 < NCH // 2)
            def _():
                _p1_start(j0 + 2, 0)

            pltpu.make_async_copy(src_hbm.at[pl.ds((j0 + 1) * K, K)],
                                  esrc.at[pl.ds(K, K)], sem_s[1]).wait()
            _p1_proc(1)
            return carry

        lax.fori_loop(0, NCH // 2, body, 0)

    _p1_pipelined()

    def _p2_proc(slot):
        def ib(i, carry):
            sl = pl.ds(slot * K + i * 16, 16)
            s = esrc[sl]
            d = edst[sl]
            wv = ew[sl]
            c = wv / plsc.load_gather(deg_v, [s])
            plsc.addupdate_scatter(s_v, [s], c)
            plsc.addupdate_scatter(s_v, [d], c)
            for q in range(QPW):
                off = jnp.int32(q * N)
                sq = s + off
                dq = d + off
                fd = plsc.load_gather(f_v, [dq])
                plsc.addupdate_scatter(acc_v, [sq], c * fd)
                fs = plsc.load_gather(f_v, [sq])
                plsc.addupdate_scatter(acc_v, [dq], c * fs)
            return carry

        lax.fori_loop(0, IPV, ib, 0)

    _pipelined(_start, _p2_proc)

    qb = wid * QPW
    xi_scale = jnp.float32(MAX_XI / (Q - 1))
    for q in range(QPW):
        xi_q = (qb + q).astype(jnp.float32) * xi_scale
        qoff = q * N

        def ck_body(ck, carry, xi_q=xi_q, qoff=qoff):
            hb = base + qoff + ck * K
            pltpu.sync_copy(collT.at[pl.ds(hb, K)], cb)
            pltpu.sync_copy(srcT.at[pl.ds(hb, K)], sb)

            def ib(i, cc):
                sl = pl.ds(i * 16, 16)
                lo = pl.ds(qoff + ck * K + i * 16, 16)
                ns = pl.ds(ck * K + i * 16, 16)
                fl = f_v[lo]
                tr = xi_q * (acc_v[lo] - s_v[ns] * fl)
                o = fl - DT * (tr - cb[sl] - sb[sl])
                ob[sl] = jnp.maximum(o, 0.0)
                return cc

            lax.fori_loop(0, IPV, ib, 0)
            pltpu.sync_copy(ob, out.at[pl.ds(hb, K)])
            return carry

        lax.fori_loop(0, N // K, ck_body, 0)


def kernel(f_distribution, collision_term, source_term, edge_weight, edge_index):
    fT = jnp.transpose(f_distribution).reshape(-1)
    collT = jnp.transpose(collision_term).reshape(-1)
    srcT = jnp.transpose(source_term).reshape(-1)
    src = edge_index[0].astype(jnp.int32)
    dst = edge_index[1].astype(jnp.int32)
    outT = _sc_step(fT, collT, srcT, edge_weight, src, dst)
    return jnp.transpose(outT.reshape(Q, N))
